# Initial kernel scaffold; baseline (speedup 1.0000x reference)
import jax
import jax.numpy as jnp
from jax.experimental import pallas as pl


def kernel(input_ids, W):
    raise NotImplementedError("write your pallas kernel here")


# speedup vs baseline: 5.7899x; 5.7899x over previous
import functools

import jax
import jax.numpy as jnp
from jax import lax
from jax.experimental import pallas as pl
from jax.experimental.pallas import tpu as pltpu
from jax.experimental.pallas import tpu_sc as plsc

_INFO = plsc.get_sparse_core_info()
_NC = _INFO.num_cores
_NS = _INFO.num_subcores
_NW = _NC * _NS
_CHUNK = 128


def _embed_lookup(idx_grouped, W, n_rows, d):
    k = idx_grouped.shape[1]
    per_w = k * _CHUNK

    @functools.partial(
        pl.kernel,
        mesh=plsc.VectorSubcoreMesh(core_axis_name="c", subcore_axis_name="s"),
        out_type=jax.ShapeDtypeStruct((n_rows, d), jnp.float32),
        scratch_types=[
            pltpu.VMEM((k, _CHUNK), jnp.int32),
            pltpu.VMEM((_CHUNK, d), jnp.float32),
            pltpu.SemaphoreType.DMA,
        ],
    )
    def emb(idx_hbm, table_hbm, out_hbm, idx_v, rows_v, sem):
        wid = lax.axis_index("s") * _NC + lax.axis_index("c")
        base = wid * per_w
        pltpu.sync_copy(idx_hbm.at[wid], idx_v)

        def body(j, carry):
            pltpu.async_copy(table_hbm.at[idx_v.at[j]], rows_v, sem).wait()
            pltpu.sync_copy(rows_v, out_hbm.at[pl.ds(base + j * _CHUNK, _CHUNK)])
            return carry

        lax.fori_loop(0, k, body, 0)

    return emb(idx_grouped, W)


def kernel(input_ids, W):
    B, L = input_ids.shape
    V, D = W.shape
    n = B * L
    idx = input_ids.reshape(_NW, n // (_NW * _CHUNK), _CHUNK).astype(jnp.int32)
    out = _embed_lookup(idx, W, n, D)
    return out.reshape(B, L, D)


# speedup vs baseline: 6.6135x; 1.1423x over previous
import functools

import jax
import jax.numpy as jnp
from jax import lax
from jax.experimental import pallas as pl
from jax.experimental.pallas import tpu as pltpu
from jax.experimental.pallas import tpu_sc as plsc

_INFO = plsc.get_sparse_core_info()
_NC = _INFO.num_cores
_NS = _INFO.num_subcores
_NW = _NC * _NS
_CHUNK = 128


def _embed_lookup(idx_grouped, W, n_rows, d):
    k = idx_grouped.shape[1]
    per_w = k * _CHUNK
    assert k % 2 == 0
    nm = k // 2

    @functools.partial(
        pl.kernel,
        mesh=plsc.VectorSubcoreMesh(core_axis_name="c", subcore_axis_name="s"),
        out_type=jax.ShapeDtypeStruct((n_rows, d), jnp.float32),
        scratch_types=[
            pltpu.VMEM((k, _CHUNK), jnp.int32),
            pltpu.VMEM((_CHUNK, d), jnp.float32),
            pltpu.VMEM((_CHUNK, d), jnp.float32),
            pltpu.SemaphoreType.DMA,
            pltpu.SemaphoreType.DMA,
            pltpu.SemaphoreType.DMA,
            pltpu.SemaphoreType.DMA,
        ],
    )
    def emb(idx_hbm, table_hbm, out_hbm, idx_v, buf0, buf1, gs0, gs1, ws0, ws1):
        wid = lax.axis_index("s") * _NC + lax.axis_index("c")
        base = wid * per_w
        pltpu.sync_copy(idx_hbm.at[wid], idx_v)

        def wait_gather(buf, sem):
            pltpu.make_async_copy(table_hbm.at[pl.ds(0, _CHUNK)], buf, sem).wait()

        def wait_write(buf, sem):
            pltpu.make_async_copy(buf, out_hbm.at[pl.ds(0, _CHUNK)], sem).wait()

        pltpu.async_copy(table_hbm.at[idx_v.at[0]], buf0, gs0)

        def body(m, carry):
            j0 = 2 * m
            wait_gather(buf0, gs0)

            @pl.when(m > 0)
            def _():
                wait_write(buf1, ws1)

            pltpu.async_copy(table_hbm.at[idx_v.at[j0 + 1]], buf1, gs1)
            pltpu.async_copy(buf0, out_hbm.at[pl.ds(base + j0 * _CHUNK, _CHUNK)], ws0)

            wait_gather(buf1, gs1)
            wait_write(buf0, ws0)

            @pl.when(m < nm - 1)
            def _():
                pltpu.async_copy(table_hbm.at[idx_v.at[j0 + 2]], buf0, gs0)

            pltpu.async_copy(
                buf1, out_hbm.at[pl.ds(base + (j0 + 1) * _CHUNK, _CHUNK)], ws1)
            return carry

        lax.fori_loop(0, nm, body, 0)
        wait_write(buf1, ws1)

    return emb(idx_grouped, W)


def kernel(input_ids, W):
    B, L = input_ids.shape
    V, D = W.shape
    n = B * L
    idx = input_ids.reshape(_NW, n // (_NW * _CHUNK), _CHUNK).astype(jnp.int32)
    out = _embed_lookup(idx, W, n, D)
    return out.reshape(B, L, D)


# speedup vs baseline: 8.0015x; 1.2099x over previous
import functools

import jax
import jax.numpy as jnp
from jax import lax
from jax.experimental import pallas as pl
from jax.experimental.pallas import tpu as pltpu
from jax.experimental.pallas import tpu_sc as plsc

_INFO = plsc.get_sparse_core_info()
_NC = _INFO.num_cores
_NS = _INFO.num_subcores
_NW = _NC * _NS
_CHUNK = 128
_NBUF = 5
_AHEAD = 3


def _embed_lookup(idx_grouped, W, n_rows, d):
    k = idx_grouped.shape[1]
    per_w = k * _CHUNK
    assert k % _NBUF == 0
    nm = k // _NBUF

    @functools.partial(
        pl.kernel,
        mesh=plsc.VectorSubcoreMesh(core_axis_name="c", subcore_axis_name="s"),
        out_type=jax.ShapeDtypeStruct((n_rows, d), jnp.float32),
        scratch_types=[
            pltpu.VMEM((k, _CHUNK), jnp.int32),
            pltpu.VMEM((_NBUF, _CHUNK, d), jnp.float32),
            pltpu.SemaphoreType.DMA((_NBUF,)),
            pltpu.SemaphoreType.DMA((_NBUF,)),
        ],
    )
    def emb(idx_hbm, table_hbm, out_hbm, idx_v, bufs, gsem, wsem):
        wid = lax.axis_index("s") * _NC + lax.axis_index("c")
        base = wid * per_w
        pltpu.sync_copy(idx_hbm.at[wid], idx_v)

        def start_gather(j, b):
            pltpu.async_copy(table_hbm.at[idx_v.at[j]], bufs.at[b], gsem.at[b])

        def wait_gather(b):
            pltpu.make_async_copy(
                table_hbm.at[pl.ds(0, _CHUNK)], bufs.at[b], gsem.at[b]).wait()

        def start_write(j, b):
            pltpu.async_copy(
                bufs.at[b], out_hbm.at[pl.ds(base + j * _CHUNK, _CHUNK)],
                wsem.at[b])

        def wait_write(b):
            pltpu.make_async_copy(
                bufs.at[b], out_hbm.at[pl.ds(0, _CHUNK)], wsem.at[b]).wait()

        for b in range(_AHEAD):
            start_gather(b, b)

        def body(m, carry):
            j0 = _NBUF * m
            for u in range(_NBUF):
                j = j0 + u
                pb = (u + _AHEAD) % _NBUF
                wait_gather(u)
                if u < _NBUF - _AHEAD:
                    @pl.when(m > 0)
                    def _():
                        wait_write(pb)
                        start_gather(j + _AHEAD, pb)

                    @pl.when(m == 0)
                    def _():
                        start_gather(j + _AHEAD, pb)
                else:
                    wait_write(pb)

                    @pl.when(m < nm - 1)
                    def _():
                        start_gather(j + _AHEAD, pb)
                start_write(j, u)
            return carry

        lax.fori_loop(0, nm, body, 0)
        for j in range(k - (_NBUF - _AHEAD), k):
            wait_write(j % _NBUF)

    return emb(idx_grouped, W)


def kernel(input_ids, W):
    B, L = input_ids.shape
    V, D = W.shape
    n = B * L
    idx = input_ids.reshape(_NW, n // (_NW * _CHUNK), _CHUNK).astype(jnp.int32)
    out = _embed_lookup(idx, W, n, D)
    return out.reshape(B, L, D)


# speedup vs baseline: 8.0507x; 1.0061x over previous
import functools

import jax
import jax.numpy as jnp
from jax import lax
from jax.experimental import pallas as pl
from jax.experimental.pallas import tpu as pltpu
from jax.experimental.pallas import tpu_sc as plsc

_INFO = plsc.get_sparse_core_info()
_NC = _INFO.num_cores
_NS = _INFO.num_subcores
_NW = _NC * _NS
_CHUNK = 80
_NBUF = 8
_AHEAD = 5


def _embed_lookup(idx_grouped, W, n_rows, d):
    k = idx_grouped.shape[1]
    per_w = k * _CHUNK
    assert k % _NBUF == 0
    nm = k // _NBUF

    @functools.partial(
        pl.kernel,
        mesh=plsc.VectorSubcoreMesh(core_axis_name="c", subcore_axis_name="s"),
        out_type=jax.ShapeDtypeStruct((n_rows, d), jnp.float32),
        scratch_types=[
            pltpu.VMEM((k, _CHUNK), jnp.int32),
            pltpu.VMEM((_NBUF, _CHUNK, d), jnp.float32),
            pltpu.SemaphoreType.DMA((_NBUF,)),
            pltpu.SemaphoreType.DMA((_NBUF,)),
        ],
    )
    def emb(idx_hbm, table_hbm, out_hbm, idx_v, bufs, gsem, wsem):
        wid = lax.axis_index("s") * _NC + lax.axis_index("c")
        base = wid * per_w
        pltpu.sync_copy(idx_hbm.at[wid], idx_v)

        def start_gather(j, b):
            pltpu.async_copy(table_hbm.at[idx_v.at[j]], bufs.at[b], gsem.at[b])

        def wait_gather(b):
            pltpu.make_async_copy(
                table_hbm.at[pl.ds(0, _CHUNK)], bufs.at[b], gsem.at[b]).wait()

        def start_write(j, b):
            pltpu.async_copy(
                bufs.at[b], out_hbm.at[pl.ds(base + j * _CHUNK, _CHUNK)],
                wsem.at[b])

        def wait_write(b):
            pltpu.make_async_copy(
                bufs.at[b], out_hbm.at[pl.ds(0, _CHUNK)], wsem.at[b]).wait()

        for b in range(_AHEAD):
            start_gather(b, b)

        def body(m, carry):
            j0 = _NBUF * m
            for u in range(_NBUF):
                j = j0 + u
                pb = (u + _AHEAD) % _NBUF
                wait_gather(u)
                if u < _NBUF - _AHEAD:
                    @pl.when(m > 0)
                    def _():
                        wait_write(pb)
                        start_gather(j + _AHEAD, pb)

                    @pl.when(m == 0)
                    def _():
                        start_gather(j + _AHEAD, pb)
                else:
                    wait_write(pb)

                    @pl.when(m < nm - 1)
                    def _():
                        start_gather(j + _AHEAD, pb)
                start_write(j, u)
            return carry

        lax.fori_loop(0, nm, body, 0)
        for j in range(k - (_NBUF - _AHEAD), k):
            wait_write(j % _NBUF)

    return emb(idx_grouped, W)


def kernel(input_ids, W):
    B, L = input_ids.shape
    V, D = W.shape
    n = B * L
    idx = input_ids.reshape(_NW, n // (_NW * _CHUNK), _CHUNK).astype(jnp.int32)
    out = _embed_lookup(idx, W, n, D)
    return out.reshape(B, L, D)
